# baseline (speedup 1.0000x reference)
import functools

import jax
import jax.numpy as jnp
from jax import lax
from jax.experimental import pallas as pl
from jax.experimental.pallas import tpu as pltpu
from jax.experimental.pallas import tpu_sc as plsc

POSITION_SIZE = 8192
EMBEDDING_SIZE = 128
BATCH = 4
SEQ_LEN = 8192

NUM_CORES = 2
NUM_SUBCORES = 16
NUM_WORKERS = NUM_CORES * NUM_SUBCORES
WORKERS_PER_BATCH = NUM_WORKERS // BATCH
IDS_PER_WORKER = SEQ_LEN // WORKERS_PER_BATCH
CHUNK = 128
CHUNKS_PER_WORKER = IDS_PER_WORKER // CHUNK
NBUF = 3
TAB_ROWS_PER_TILE = POSITION_SIZE // NUM_SUBCORES

_MESH = plsc.VectorSubcoreMesh(core_axis_name="c", subcore_axis_name="s")


@functools.partial(
    pl.kernel,
    mesh=_MESH,
    out_type=jax.ShapeDtypeStruct((BATCH, SEQ_LEN, EMBEDDING_SIZE), jnp.float32),
    scratch_types=[
        pltpu.VMEM((IDS_PER_WORKER,), jnp.int32),
        pltpu.VMEM((NBUF, CHUNK, EMBEDDING_SIZE), jnp.float32),
        pltpu.VMEM_SHARED((POSITION_SIZE, EMBEDDING_SIZE), jnp.float32),
        pltpu.SemaphoreType.DMA,
        pltpu.SemaphoreType.DMA,
    ],
)
def _gather_kernel(idx_hbm, table_hbm, out_hbm, idx_v, rows_v, tab_sh,
                   gsem, osem):
    sid = lax.axis_index("s")
    wid = sid * NUM_CORES + lax.axis_index("c")
    b = wid // WORKERS_PER_BATCH
    s0 = (wid % WORKERS_PER_BATCH) * IDS_PER_WORKER

    t0 = sid * TAB_ROWS_PER_TILE
    pltpu.sync_copy(table_hbm.at[pl.ds(t0, TAB_ROWS_PER_TILE)],
                    tab_sh.at[pl.ds(t0, TAB_ROWS_PER_TILE)])
    pltpu.sync_copy(idx_hbm.at[b, pl.ds(s0, IDS_PER_WORKER)], idx_v)
    plsc.subcore_barrier()

    gathers = [None] * CHUNKS_PER_WORKER
    outs = [None] * CHUNKS_PER_WORKER
    for c in range(min(NBUF, CHUNKS_PER_WORKER)):
        gathers[c] = pltpu.async_copy(
            tab_sh.at[idx_v.at[pl.ds(c * CHUNK, CHUNK)]],
            rows_v.at[c % NBUF], gsem)
    for c in range(CHUNKS_PER_WORKER):
        gathers[c].wait()
        outs[c] = pltpu.async_copy(
            rows_v.at[c % NBUF],
            out_hbm.at[b, pl.ds(s0 + c * CHUNK, CHUNK)],
            osem)
        nxt = c + NBUF
        if nxt < CHUNKS_PER_WORKER:
            outs[c].wait()
            gathers[nxt] = pltpu.async_copy(
                tab_sh.at[idx_v.at[pl.ds(nxt * CHUNK, CHUNK)]],
                rows_v.at[nxt % NBUF], gsem)
    for c in range(max(0, CHUNKS_PER_WORKER - NBUF), CHUNKS_PER_WORKER):
        outs[c].wait()


def kernel(input_ids, embedding_table):
    out = _gather_kernel(input_ids, embedding_table)
    return out, embedding_table
